# baseline (speedup 1.0000x reference)
import functools

import jax
import jax.numpy as jnp
from jax.experimental import pallas as pl
from jax.experimental.pallas import tpu as pltpu

INPUT_DIM = 256
D_U = 256
KNN_K = 4
B, T = 8, 512
FEAT_NOSEL = INPUT_DIM + 3
FEAT_DIM = FEAT_NOSEL + 1
CTX_DIM = 2 * INPUT_DIM
IN_DIM = FEAT_DIM + CTX_DIM

_HIGHEST = jax.lax.Precision.HIGHEST
_PREC = jax.lax.Precision.DEFAULT
_INT_INF = 2**31 - 1


def _encoder_kernel(x_ref, ln_g_ref, ln_b_ref, w1_ref, b1_ref, w2_ref,
                    b2_ref, u_ref, sf_ref, sim_ref, ctx_ref, prep_ref):
    @pl.when(pl.program_id(0) == 0)
    def _prep():
        w1 = w1_ref[...]
        lg = ln_g_ref[...].reshape(1, IN_DIM)
        lb = ln_b_ref[...].reshape(1, IN_DIM)
        prep_ref[0:1] = jax.lax.dot_general(
            lg, w1, (((1,), (0,)), ((), ())),
            preferred_element_type=jnp.float32, precision=_PREC)
        prep_ref[1:2] = jax.lax.dot_general(
            lb, w1, (((1,), (0,)), ((), ())),
            preferred_element_type=jnp.float32, precision=_PREC) + b1_ref[...]

    xb = x_ref[0]

    sq = jnp.sum(xb * xb, axis=1, keepdims=True)
    gram = jax.lax.dot_general(
        xb, xb, (((1,), (1,)), ((), ())),
        preferred_element_type=jnp.float32, precision=_PREC)
    E = sq - 2.0 * gram
    sqrow = sq.T
    vprev = jnp.min(E, axis=0, keepdims=True)
    acc = jnp.zeros((1, T), jnp.float32)
    for _ in range(KNN_K):
        vprev = jnp.min(jnp.where(E > vprev, E, 1e18),
                        axis=0, keepdims=True)
        d2v = jnp.maximum(vprev + sqrow, 0.0)
        acc = acc + jnp.sqrt(d2v + 1e-12)
    knn_mean = (acc * (1.0 / KNN_K)).T

    mu_t = jnp.mean(xb, axis=0, keepdims=True)
    diff = xb - mu_t
    cdist = jnp.sqrt(jnp.sum(diff * diff, axis=1, keepdims=True) + 1e-12)
    nrm = jnp.sqrt(sq + 1e-12)
    var_t = jnp.mean(diff * diff, axis=0, keepdims=True)
    sd_t = jnp.sqrt(var_t + 1e-6)
    ctx = jnp.concatenate([mu_t, sd_t], axis=1)
    ctx_ref[pl.ds(pl.program_id(0), 1), :] = ctx

    ones = jnp.ones((T, 1), jnp.float32)
    sf = jnp.concatenate([xb, cdist, knn_mean, nrm, ones], axis=1)
    sf_ref[0] = sf

    s_ctx = jnp.sum(ctx, axis=1, keepdims=True)
    s2_ctx = jnp.sum(ctx * ctx, axis=1, keepdims=True)
    mu_h = (jnp.sum(sf, axis=1, keepdims=True) + s_ctx) * (1.0 / IN_DIM)
    ex2 = (jnp.sum(sf * sf, axis=1, keepdims=True) + s2_ctx) * (1.0 / IN_DIM)
    inv_sd = jax.lax.rsqrt(jnp.maximum(ex2 - mu_h * mu_h, 0.0) + 1e-5)

    g = ln_g_ref[...].reshape(1, IN_DIM)
    sfg = sf * g[:, :FEAT_DIM]
    ctxg = ctx * g[:, FEAT_DIM:]
    w1 = w1_ref[...]
    core = jax.lax.dot_general(
        sfg, w1[:FEAT_DIM], (((1,), (0,)), ((), ())),
        preferred_element_type=jnp.float32, precision=_PREC)
    ctxw = jax.lax.dot_general(
        ctxg, w1[FEAT_DIM:], (((1,), (0,)), ((), ())),
        preferred_element_type=jnp.float32, precision=_PREC)
    colsum = prep_ref[0:1]
    cvec = prep_ref[1:2]
    h1 = inv_sd * (core + ctxw) - (mu_h * inv_sd) * colsum + cvec
    h1 = 0.5 * h1 * (1.0 + jax.lax.erf(h1 * 0.7071067811865476))
    u = jax.lax.dot_general(
        h1, w2_ref[...], (((1,), (0,)), ((), ())),
        preferred_element_type=jnp.float32, precision=_PREC) + b2_ref[...]
    u_ref[0] = u

    rowsq = sq + cdist * cdist + knn_mean * knn_mean + nrm * nrm
    inv = 1.0 / (jnp.sqrt(rowsq) + 1e-8)
    ff = gram + cdist * cdist.T + knn_mean * knn_mean.T + nrm * nrm.T
    sim_ref[0] = (inv * inv.T) * ff


@functools.partial(jax.jit, static_argnames=())
def kernel(x, ln_g, ln_b, W1, b1, W2, b2):

    rep = lambda *shape: pl.BlockSpec(shape, lambda b: (0,) * len(shape))
    out_shapes = (
        jax.ShapeDtypeStruct((B, T, D_U), jnp.float32),
        jax.ShapeDtypeStruct((B, T, FEAT_DIM), jnp.float32),
        jax.ShapeDtypeStruct((B, T, T), jnp.float32),
        jax.ShapeDtypeStruct((B, CTX_DIM), jnp.float32),
    )
    u, sf, sim, ctx = pl.pallas_call(
        _encoder_kernel,
        grid=(B,),
        in_specs=[
            pl.BlockSpec((1, T, INPUT_DIM), lambda b: (b, 0, 0)),
            rep(IN_DIM),
            rep(IN_DIM),
            rep(IN_DIM, D_U),
            rep(D_U),
            rep(D_U, D_U),
            rep(D_U),
        ],
        out_specs=(
            pl.BlockSpec((1, T, D_U), lambda b: (b, 0, 0)),
            pl.BlockSpec((1, T, FEAT_DIM), lambda b: (b, 0, 0)),
            pl.BlockSpec((1, T, T), lambda b: (b, 0, 0)),
            pl.BlockSpec((B, CTX_DIM), lambda b: (0, 0)),
        ),
        out_shape=out_shapes,
        scratch_shapes=[pltpu.VMEM((2, D_U), jnp.float32)],
    )(x, ln_g, ln_b, W1, b1, W2, b2)
    return (u, sf, sim, ctx)


# speedup vs baseline: 1.0635x; 1.0635x over previous
import functools

import jax
import jax.numpy as jnp
from jax.experimental import pallas as pl
from jax.experimental.pallas import tpu as pltpu

INPUT_DIM = 256
D_U = 256
KNN_K = 4
B, T = 8, 512
FEAT_NOSEL = INPUT_DIM + 3
FEAT_DIM = FEAT_NOSEL + 1
CTX_DIM = 2 * INPUT_DIM
IN_DIM = FEAT_DIM + CTX_DIM

_HIGHEST = jax.lax.Precision.HIGHEST
_PREC = jax.lax.Precision.DEFAULT
_INT_INF = 2**31 - 1


def _encoder_kernel(x_ref, ln_g_ref, ln_b_ref, w1_ref, b1_ref, w2_ref,
                    b2_ref, u_ref, sf_ref, sim_ref, ctx_ref, prep_ref):
    @pl.when(pl.program_id(0) == 0)
    def _prep():
        w1 = w1_ref[...]
        lg = ln_g_ref[...].reshape(1, IN_DIM)
        lb = ln_b_ref[...].reshape(1, IN_DIM)
        prep_ref[0:1] = jax.lax.dot_general(
            lg, w1, (((1,), (0,)), ((), ())),
            preferred_element_type=jnp.float32, precision=_PREC)
        prep_ref[1:2] = jax.lax.dot_general(
            lb, w1, (((1,), (0,)), ((), ())),
            preferred_element_type=jnp.float32, precision=_PREC) + b1_ref[...]

    xb = x_ref[0]

    sq = jnp.sum(xb * xb, axis=1, keepdims=True)
    gram = jax.lax.dot_general(
        xb, xb, (((1,), (1,)), ((), ())),
        preferred_element_type=jnp.float32, precision=_PREC)
    row = jax.lax.broadcasted_iota(jnp.int32, (T, T), 0)
    col = jax.lax.broadcasted_iota(jnp.int32, (T, T), 1)
    E = jnp.where(row == col, 1e18, sq - 2.0 * gram)
    sqrow = sq.T
    vprev = jnp.min(E, axis=0, keepdims=True)
    acc = jnp.sqrt(jnp.maximum(vprev + sqrow, 0.0) + 1e-12)
    for _ in range(KNN_K - 1):
        vprev = jnp.min(jnp.where(E > vprev, E, 1e18),
                        axis=0, keepdims=True)
        acc = acc + jnp.sqrt(jnp.maximum(vprev + sqrow, 0.0) + 1e-12)
    knn_mean = (acc * (1.0 / KNN_K)).T

    mu_t = jnp.mean(xb, axis=0, keepdims=True)
    diff = xb - mu_t
    cdist = jnp.sqrt(jnp.sum(diff * diff, axis=1, keepdims=True) + 1e-12)
    nrm = jnp.sqrt(sq + 1e-12)
    var_t = jnp.mean(diff * diff, axis=0, keepdims=True)
    sd_t = jnp.sqrt(var_t + 1e-6)
    ctx = jnp.concatenate([mu_t, sd_t], axis=1)
    ctx_ref[pl.ds(pl.program_id(0), 1), :] = ctx

    ones = jnp.ones((T, 1), jnp.float32)
    sf = jnp.concatenate([xb, cdist, knn_mean, nrm, ones], axis=1)
    sf_ref[0] = sf

    s_ctx = jnp.sum(ctx, axis=1, keepdims=True)
    s2_ctx = jnp.sum(ctx * ctx, axis=1, keepdims=True)
    mu_h = (jnp.sum(sf, axis=1, keepdims=True) + s_ctx) * (1.0 / IN_DIM)
    ex2 = (jnp.sum(sf * sf, axis=1, keepdims=True) + s2_ctx) * (1.0 / IN_DIM)
    inv_sd = jax.lax.rsqrt(jnp.maximum(ex2 - mu_h * mu_h, 0.0) + 1e-5)

    g = ln_g_ref[...].reshape(1, IN_DIM)
    sfg = sf * g[:, :FEAT_DIM]
    ctxg = ctx * g[:, FEAT_DIM:]
    w1 = w1_ref[...]
    core = jax.lax.dot_general(
        sfg, w1[:FEAT_DIM], (((1,), (0,)), ((), ())),
        preferred_element_type=jnp.float32, precision=_PREC)
    ctxw = jax.lax.dot_general(
        ctxg, w1[FEAT_DIM:], (((1,), (0,)), ((), ())),
        preferred_element_type=jnp.float32, precision=_PREC)
    colsum = prep_ref[0:1]
    cvec = prep_ref[1:2]
    h1 = inv_sd * (core + ctxw) - (mu_h * inv_sd) * colsum + cvec
    h1 = 0.5 * h1 * (1.0 + jax.lax.erf(h1 * 0.7071067811865476))
    u = jax.lax.dot_general(
        h1, w2_ref[...], (((1,), (0,)), ((), ())),
        preferred_element_type=jnp.float32, precision=_PREC) + b2_ref[...]
    u_ref[0] = u

    rowsq = sq + cdist * cdist + knn_mean * knn_mean + nrm * nrm
    inv = 1.0 / (jnp.sqrt(rowsq) + 1e-8)
    ff = gram + cdist * cdist.T + knn_mean * knn_mean.T + nrm * nrm.T
    sim_ref[0] = (inv * inv.T) * ff


@functools.partial(jax.jit, static_argnames=())
def kernel(x, ln_g, ln_b, W1, b1, W2, b2):

    rep = lambda *shape: pl.BlockSpec(shape, lambda b: (0,) * len(shape))
    out_shapes = (
        jax.ShapeDtypeStruct((B, T, D_U), jnp.float32),
        jax.ShapeDtypeStruct((B, T, FEAT_DIM), jnp.float32),
        jax.ShapeDtypeStruct((B, T, T), jnp.float32),
        jax.ShapeDtypeStruct((B, CTX_DIM), jnp.float32),
    )
    u, sf, sim, ctx = pl.pallas_call(
        _encoder_kernel,
        grid=(B,),
        in_specs=[
            pl.BlockSpec((1, T, INPUT_DIM), lambda b: (b, 0, 0)),
            rep(IN_DIM),
            rep(IN_DIM),
            rep(IN_DIM, D_U),
            rep(D_U),
            rep(D_U, D_U),
            rep(D_U),
        ],
        out_specs=(
            pl.BlockSpec((1, T, D_U), lambda b: (b, 0, 0)),
            pl.BlockSpec((1, T, FEAT_DIM), lambda b: (b, 0, 0)),
            pl.BlockSpec((1, T, T), lambda b: (b, 0, 0)),
            pl.BlockSpec((B, CTX_DIM), lambda b: (0, 0)),
        ),
        out_shape=out_shapes,
        scratch_shapes=[pltpu.VMEM((2, D_U), jnp.float32)],
    )(x, ln_g, ln_b, W1, b1, W2, b2)
    return (u, sf, sim, ctx)


# speedup vs baseline: 1.0677x; 1.0040x over previous
import functools

import jax
import jax.numpy as jnp
from jax.experimental import pallas as pl
from jax.experimental.pallas import tpu as pltpu

INPUT_DIM = 256
D_U = 256
KNN_K = 4
B, T = 8, 512
FEAT_NOSEL = INPUT_DIM + 3
FEAT_DIM = FEAT_NOSEL + 1
CTX_DIM = 2 * INPUT_DIM
IN_DIM = FEAT_DIM + CTX_DIM

_PREC = jax.lax.Precision.DEFAULT


def _encoder_kernel(x_ref, ln_g_ref, ln_b_ref, w1_ref, b1_ref, w2_ref,
                    b2_ref, u_ref, sf_ref, sim_ref, ctx_ref, prep_ref):
    @pl.when(pl.program_id(0) == 0)
    def _prep():
        w1 = w1_ref[...]
        lg = ln_g_ref[...].reshape(1, IN_DIM)
        lb = ln_b_ref[...].reshape(1, IN_DIM)
        prep_ref[0:1] = jax.lax.dot_general(
            lg, w1, (((1,), (0,)), ((), ())),
            preferred_element_type=jnp.float32, precision=_PREC)
        prep_ref[1:2] = jax.lax.dot_general(
            lb, w1, (((1,), (0,)), ((), ())),
            preferred_element_type=jnp.float32, precision=_PREC) + b1_ref[...]

    xb = x_ref[0]

    sq = jnp.sum(xb * xb, axis=1, keepdims=True)
    gram = jax.lax.dot_general(
        xb, xb, (((1,), (1,)), ((), ())),
        preferred_element_type=jnp.float32, precision=_PREC)
    row = jax.lax.broadcasted_iota(jnp.int32, (T, T), 0)
    col = jax.lax.broadcasted_iota(jnp.int32, (T, T), 1)
    E = jnp.where(row == col, 1e18, sq - 2.0 * gram)
    sqrow = sq.T
    vprev = jnp.min(E, axis=0, keepdims=True)
    acc = jnp.sqrt(jnp.maximum(vprev + sqrow, 0.0) + 1e-12)
    for _ in range(KNN_K - 1):
        vprev = jnp.min(jnp.where(E > vprev, E, 1e18),
                        axis=0, keepdims=True)
        acc = acc + jnp.sqrt(jnp.maximum(vprev + sqrow, 0.0) + 1e-12)
    knn_mean = (acc * (1.0 / KNN_K)).T

    mu_t = jnp.mean(xb, axis=0, keepdims=True)
    diff = xb - mu_t
    cdist = jnp.sqrt(jnp.sum(diff * diff, axis=1, keepdims=True) + 1e-12)
    nrm = jnp.sqrt(sq + 1e-12)
    var_t = jnp.mean(diff * diff, axis=0, keepdims=True)
    sd_t = jnp.sqrt(var_t + 1e-6)
    ctx = jnp.concatenate([mu_t, sd_t], axis=1)
    ctx_ref[pl.ds(pl.program_id(0), 1), :] = ctx

    ones = jnp.ones((T, 1), jnp.float32)
    sf = jnp.concatenate([xb, cdist, knn_mean, nrm, ones], axis=1)
    sf_ref[0] = sf

    s_ctx = jnp.sum(ctx, axis=1, keepdims=True)
    s2_ctx = jnp.sum(ctx * ctx, axis=1, keepdims=True)
    mu_h = (jnp.sum(sf, axis=1, keepdims=True) + s_ctx) * (1.0 / IN_DIM)
    ex2 = (jnp.sum(sf * sf, axis=1, keepdims=True) + s2_ctx) * (1.0 / IN_DIM)
    inv_sd = jax.lax.rsqrt(jnp.maximum(ex2 - mu_h * mu_h, 0.0) + 1e-5)

    g = ln_g_ref[...].reshape(1, IN_DIM)
    sfg = sf * g[:, :FEAT_DIM]
    ctxg = ctx * g[:, FEAT_DIM:]
    w1 = w1_ref[...]
    core = jax.lax.dot_general(
        sfg, w1[:FEAT_DIM], (((1,), (0,)), ((), ())),
        preferred_element_type=jnp.float32, precision=_PREC)
    ctxw = jax.lax.dot_general(
        ctxg, w1[FEAT_DIM:], (((1,), (0,)), ((), ())),
        preferred_element_type=jnp.float32, precision=_PREC)
    colsum = prep_ref[0:1]
    cvec = prep_ref[1:2]
    h1 = inv_sd * (core + ctxw) - (mu_h * inv_sd) * colsum + cvec
    h1 = 0.5 * h1 * (1.0 + jax.lax.erf(h1 * 0.7071067811865476))
    u = jax.lax.dot_general(
        h1, w2_ref[...], (((1,), (0,)), ((), ())),
        preferred_element_type=jnp.float32, precision=_PREC) + b2_ref[...]
    u_ref[0] = u

    rowsq = sq + cdist * cdist + knn_mean * knn_mean + nrm * nrm
    inv = 1.0 / (jnp.sqrt(rowsq) + 1e-8)
    ff = gram + cdist * cdist.T + knn_mean * knn_mean.T + nrm * nrm.T
    sim_ref[0] = (inv * inv.T) * ff


@functools.partial(jax.jit, static_argnames=())
def kernel(x, ln_g, ln_b, W1, b1, W2, b2):

    rep = lambda *shape: pl.BlockSpec(shape, lambda b: (0,) * len(shape))
    out_shapes = (
        jax.ShapeDtypeStruct((B, T, D_U), jnp.float32),
        jax.ShapeDtypeStruct((B, T, FEAT_DIM), jnp.float32),
        jax.ShapeDtypeStruct((B, T, T), jnp.float32),
        jax.ShapeDtypeStruct((B, CTX_DIM), jnp.float32),
    )
    u, sf, sim, ctx = pl.pallas_call(
        _encoder_kernel,
        grid=(B,),
        in_specs=[
            pl.BlockSpec((1, T, INPUT_DIM), lambda b: (b, 0, 0)),
            rep(IN_DIM),
            rep(IN_DIM),
            rep(IN_DIM, D_U),
            rep(D_U),
            rep(D_U, D_U),
            rep(D_U),
        ],
        out_specs=(
            pl.BlockSpec((1, T, D_U), lambda b: (b, 0, 0)),
            pl.BlockSpec((1, T, FEAT_DIM), lambda b: (b, 0, 0)),
            pl.BlockSpec((1, T, T), lambda b: (b, 0, 0)),
            pl.BlockSpec((B, CTX_DIM), lambda b: (0, 0)),
        ),
        out_shape=out_shapes,
        scratch_shapes=[pltpu.VMEM((2, D_U), jnp.float32)],
    )(x, ln_g, ln_b, W1, b1, W2, b2)
    return (u, sf, sim, ctx)
